# Initial kernel scaffold; baseline (speedup 1.0000x reference)
import functools

import jax
import jax.numpy as jnp
from jax import lax
from jax.experimental import pallas as pl
from jax.experimental.pallas import tpu as pltpu
from jax.experimental.pallas import tpu_sc as plsc

HIDDEN = 128
NC = 2
NS = 16
NW = NC * NS
CHUNK = 80


def _ln(h, g, b):
    mu = jnp.mean(h, axis=-1, keepdims=True)
    d = h - mu
    var = jnp.mean(d * d, axis=-1, keepdims=True)
    return d * lax.rsqrt(var + 1e-5) * g + b


def _dot(a, b):
    return jnp.dot(a, b, preferred_element_type=jnp.float32)


def _grid_body(x_ref, we1s_ref, wg1_ref, bg1_ref, wg2_ref, bg2_ref, gg_ref,
               bbg_ref, p_ref, out_ref):
    x = x_ref[...]
    p_ref[...] = _dot(x, we1s_ref[...])
    h = _dot(x, wg1_ref[...]) + bg1_ref[...]
    h = h * jax.nn.sigmoid(h)
    h2 = _dot(h, wg2_ref[...]) + bg2_ref[...]
    out_ref[...] = x + _ln(h2, gg_ref[...], bbg_ref[...])


def _proj_body(m_ref, we1d_ref, p_ref):
    p_ref[...] = _dot(m_ref[...], we1d_ref[...])


def _edge_body(e_ref, s1_ref, s2_ref, we1e_ref, be1_ref, we2_ref, be2_ref,
               ge_ref, bbe_ref, out_ref):
    e = e_ref[...]
    h = _dot(e, we1e_ref[...]) + s1_ref[...] + s2_ref[...] + be1_ref[...]
    h = h * jax.nn.sigmoid(h)
    h2 = _dot(h, we2_ref[...]) + be2_ref[...]
    out_ref[...] = e + _ln(h2, ge_ref[...], bbe_ref[...])


def _node_body(a1_ref, a2_ref, m_ref, wn1a_ref, wn1m_ref, bn1_ref, wn2_ref,
               bn2_ref, gn_ref, bbn_ref, out_ref):
    m = m_ref[...]
    agg = a1_ref[...] + a2_ref[...]
    h = _dot(agg, wn1a_ref[...]) + _dot(m, wn1m_ref[...]) + bn1_ref[...]
    h = h * jax.nn.sigmoid(h)
    h2 = _dot(h, wn2_ref[...]) + bn2_ref[...]
    out_ref[...] = m + _ln(h2, gn_ref[...], bbn_ref[...])


def _full(shape):
    return pl.BlockSpec(shape, lambda i: (0,) * len(shape))


def _rows(br):
    return pl.BlockSpec((br, HIDDEN), lambda i: (i, 0))


def _sc_mesh():
    return plsc.VectorSubcoreMesh(core_axis_name="c", subcore_axis_name="s",
                                  num_cores=NC, num_subcores=NS)


def _make_gather(n_edge, n_chunks):
    @functools.partial(
        pl.kernel,
        out_type=[jax.ShapeDtypeStruct((n_edge, HIDDEN), jnp.float32),
                  jax.ShapeDtypeStruct((n_edge, HIDDEN), jnp.float32)],
        mesh=_sc_mesh(),
        scratch_types=[
            pltpu.VMEM((CHUNK,), jnp.int32),
            pltpu.VMEM((CHUNK,), jnp.int32),
            pltpu.VMEM((CHUNK, HIDDEN), jnp.float32),
            pltpu.VMEM((CHUNK, HIDDEN), jnp.float32),
            pltpu.SemaphoreType.DMA,
            pltpu.SemaphoreType.DMA,
        ],
    )
    def gather_k(psrc_hbm, pdst_hbm, srci_hbm, dsti_hbm, s1_hbm, s2_hbm,
                 idx1, idx2, rows1, rows2, sem1, sem2):
        wid = lax.axis_index("s") * NC + lax.axis_index("c")
        base0 = wid * (n_chunks * CHUNK)

        def body(i, carry):
            base = base0 + i * CHUNK
            pltpu.sync_copy(srci_hbm.at[pl.ds(base, CHUNK)], idx1)
            pltpu.sync_copy(dsti_hbm.at[pl.ds(base, CHUNK)], idx2)
            g1 = pltpu.async_copy(psrc_hbm.at[idx1], rows1, sem1)
            g2 = pltpu.async_copy(pdst_hbm.at[idx2], rows2, sem2)
            g1.wait()
            g2.wait()
            pltpu.sync_copy(rows1, s1_hbm.at[pl.ds(base, CHUNK)])
            pltpu.sync_copy(rows2, s2_hbm.at[pl.ds(base, CHUNK)])
            return carry

        lax.fori_loop(0, n_chunks, body, 0)

    return gather_k


def _make_scatter(n_edge, n_mesh, n_chunks):
    rows_per_tile = n_mesh // NS

    @functools.partial(
        pl.kernel,
        out_type=jax.ShapeDtypeStruct((NC * n_mesh, HIDDEN), jnp.float32),
        mesh=_sc_mesh(),
        scratch_types=[
            pltpu.VMEM((CHUNK,), jnp.int32),
            pltpu.VMEM((CHUNK, HIDDEN), jnp.float32),
            pltpu.VMEM((rows_per_tile, HIDDEN), jnp.float32),
            pltpu.VMEM_SHARED((n_mesh, HIDDEN), jnp.float32),
        ],
    )
    def scatter_k(e_hbm, dsti_hbm, zeros_hbm, out_hbm, idx, rows, bounce, acc):
        cid = lax.axis_index("c")
        sid = lax.axis_index("s")
        wid = sid * NC + cid
        base0 = wid * (n_chunks * CHUNK)
        my_rows = pl.ds(sid * rows_per_tile, rows_per_tile)

        pltpu.sync_copy(zeros_hbm, bounce)
        pltpu.sync_copy(bounce, acc.at[my_rows])
        plsc.subcore_barrier()

        def body(i, carry):
            base = base0 + i * CHUNK
            pltpu.sync_copy(dsti_hbm.at[pl.ds(base, CHUNK)], idx)
            pltpu.sync_copy(e_hbm.at[pl.ds(base, CHUNK)], rows)
            pltpu.sync_copy(rows, acc.at[idx], add=True)
            return carry

        lax.fori_loop(0, n_chunks, body, 0)
        plsc.subcore_barrier()

        pltpu.sync_copy(acc.at[my_rows], bounce)
        pltpu.sync_copy(
            bounce, out_hbm.at[pl.ds(cid * n_mesh + sid * rows_per_tile,
                                     rows_per_tile)])

    return scatter_k


def kernel(grid_node_features, mesh_node_features, grid2mesh_edge_features,
           grid2mesh_edge_indices_src, grid2mesh_edge_indices_dst,
           We1, be1, We2, be2, ge, bbe,
           Wn1, bn1, Wn2, bn2, gn, bbn,
           Wg1, bg1, Wg2, bg2, gg, bbg):
    n_grid, hid = grid_node_features.shape
    n_mesh = mesh_node_features.shape[0]
    n_edge = grid2mesh_edge_features.shape[0]
    assert hid == HIDDEN
    assert n_edge % (NW * CHUNK) == 0 and n_mesh % NS == 0
    n_chunks = n_edge // (NW * CHUNK)

    We1_e, We1_s, We1_d = We1[:hid], We1[hid:2 * hid], We1[2 * hid:]
    Wn1_a, Wn1_m = Wn1[:hid], Wn1[hid:]
    r = lambda v: v.reshape(1, hid)

    br_g = 2000
    psrc, grid_out = pl.pallas_call(
        _grid_body,
        grid=(n_grid // br_g,),
        in_specs=[_rows(br_g), _full((hid, hid)), _full((hid, hid)),
                  _full((1, hid)), _full((hid, hid)), _full((1, hid)),
                  _full((1, hid)), _full((1, hid))],
        out_specs=[_rows(br_g), _rows(br_g)],
        out_shape=[jax.ShapeDtypeStruct((n_grid, hid), jnp.float32),
                   jax.ShapeDtypeStruct((n_grid, hid), jnp.float32)],
    )(grid_node_features, We1_s, Wg1, r(bg1), Wg2, r(bg2), r(gg), r(bbg))

    br_m = 2500
    pdst = pl.pallas_call(
        _proj_body,
        grid=(n_mesh // br_m,),
        in_specs=[_rows(br_m), _full((hid, hid))],
        out_specs=_rows(br_m),
        out_shape=jax.ShapeDtypeStruct((n_mesh, hid), jnp.float32),
    )(mesh_node_features, We1_d)

    s1, s2 = _make_gather(n_edge, n_chunks)(
        psrc, pdst, grid2mesh_edge_indices_src, grid2mesh_edge_indices_dst)

    br_e = 2000
    e_feats = pl.pallas_call(
        _edge_body,
        grid=(n_edge // br_e,),
        in_specs=[_rows(br_e), _rows(br_e), _rows(br_e), _full((hid, hid)),
                  _full((1, hid)), _full((hid, hid)), _full((1, hid)),
                  _full((1, hid)), _full((1, hid))],
        out_specs=_rows(br_e),
        out_shape=jax.ShapeDtypeStruct((n_edge, hid), jnp.float32),
    )(grid2mesh_edge_features, s1, s2, We1_e, r(be1), We2, r(be2), r(ge),
      r(bbe))

    zeros_tile = jnp.zeros((n_mesh // NS, hid), jnp.float32)
    partials = _make_scatter(n_edge, n_mesh, n_chunks)(
        e_feats, grid2mesh_edge_indices_dst, zeros_tile)

    br_n = 2500
    a_spec = pl.BlockSpec((br_n, hid), lambda i: (i, 0))
    b_spec = pl.BlockSpec((br_n, hid),
                          lambda i: (i + n_mesh // br_n, 0))
    mesh_out = pl.pallas_call(
        _node_body,
        grid=(n_mesh // br_n,),
        in_specs=[a_spec, b_spec, _rows(br_n), _full((hid, hid)),
                  _full((hid, hid)), _full((1, hid)), _full((hid, hid)),
                  _full((1, hid)), _full((1, hid)), _full((1, hid))],
        out_specs=_rows(br_n),
        out_shape=jax.ShapeDtypeStruct((n_mesh, hid), jnp.float32),
    )(partials, partials, mesh_node_features, Wn1_a, Wn1_m, r(bn1), Wn2,
      r(bn2), r(gn), r(bbn))

    return (grid_out, mesh_out)


# speedup vs baseline: 2.7108x; 2.7108x over previous
import functools

import jax
import jax.numpy as jnp
from jax import lax
from jax.experimental import pallas as pl
from jax.experimental.pallas import tpu as pltpu
from jax.experimental.pallas import tpu_sc as plsc

HIDDEN = 128
NC = 2
NS = 16
NW = NC * NS
CHUNK = 80


def _ln(h, g, b):
    mu = jnp.mean(h, axis=-1, keepdims=True)
    d = h - mu
    var = jnp.mean(d * d, axis=-1, keepdims=True)
    return d * lax.rsqrt(var + 1e-5) * g + b


def _dot(a, b):
    return jnp.dot(a, b, preferred_element_type=jnp.float32)


def _grid_body(x_ref, we1s_ref, wg1_ref, bg1_ref, wg2_ref, bg2_ref, gg_ref,
               bbg_ref, p_ref, out_ref):
    x = x_ref[...]
    p_ref[...] = _dot(x, we1s_ref[...])
    h = _dot(x, wg1_ref[...]) + bg1_ref[...]
    h = h * jax.nn.sigmoid(h)
    h2 = _dot(h, wg2_ref[...]) + bg2_ref[...]
    out_ref[...] = x + _ln(h2, gg_ref[...], bbg_ref[...])


def _proj_body(m_ref, we1d_ref, p_ref):
    p_ref[...] = _dot(m_ref[...], we1d_ref[...])


def _edge_body(e_ref, s1_ref, s2_ref, we1e_ref, be1_ref, we2_ref, be2_ref,
               ge_ref, bbe_ref, out_ref):
    e = e_ref[...]
    h = _dot(e, we1e_ref[...]) + s1_ref[...] + s2_ref[...] + be1_ref[...]
    h = h * jax.nn.sigmoid(h)
    h2 = _dot(h, we2_ref[...]) + be2_ref[...]
    out_ref[...] = e + _ln(h2, ge_ref[...], bbe_ref[...])


def _node_body(a1_ref, a2_ref, m_ref, wn1a_ref, wn1m_ref, bn1_ref, wn2_ref,
               bn2_ref, gn_ref, bbn_ref, out_ref):
    m = m_ref[...]
    agg = a1_ref[...] + a2_ref[...]
    h = _dot(agg, wn1a_ref[...]) + _dot(m, wn1m_ref[...]) + bn1_ref[...]
    h = h * jax.nn.sigmoid(h)
    h2 = _dot(h, wn2_ref[...]) + bn2_ref[...]
    out_ref[...] = m + _ln(h2, gn_ref[...], bbn_ref[...])


def _full(shape):
    return pl.BlockSpec(shape, lambda i: (0,) * len(shape))


def _rows(br):
    return pl.BlockSpec((br, HIDDEN), lambda i: (i, 0))


def _sc_mesh():
    return plsc.VectorSubcoreMesh(core_axis_name="c", subcore_axis_name="s",
                                  num_cores=NC, num_subcores=NS)


def _make_gather(n_edge, n_chunks):
    @functools.partial(
        pl.kernel,
        out_type=[jax.ShapeDtypeStruct((n_edge, HIDDEN), jnp.float32),
                  jax.ShapeDtypeStruct((n_edge, HIDDEN), jnp.float32)],
        mesh=_sc_mesh(),
        scratch_types=[
            pltpu.VMEM((CHUNK,), jnp.int32),
            pltpu.VMEM((CHUNK,), jnp.int32),
            pltpu.VMEM((CHUNK, HIDDEN), jnp.float32),
            pltpu.VMEM((CHUNK, HIDDEN), jnp.float32),
            pltpu.SemaphoreType.DMA,
            pltpu.SemaphoreType.DMA,
        ],
    )
    def gather_k(psrc_hbm, pdst_hbm, srci_hbm, dsti_hbm, s1_hbm, s2_hbm,
                 idx1, idx2, rows1, rows2, sem1, sem2):
        wid = lax.axis_index("s") * NC + lax.axis_index("c")
        base0 = wid * (n_chunks * CHUNK)

        def body(i, carry):
            base = base0 + i * CHUNK
            pltpu.sync_copy(srci_hbm.at[pl.ds(base, CHUNK)], idx1)
            pltpu.sync_copy(dsti_hbm.at[pl.ds(base, CHUNK)], idx2)
            g1 = pltpu.async_copy(psrc_hbm.at[idx1], rows1, sem1)
            g2 = pltpu.async_copy(pdst_hbm.at[idx2], rows2, sem2)
            g1.wait()
            g2.wait()
            pltpu.sync_copy(rows1, s1_hbm.at[pl.ds(base, CHUNK)])
            pltpu.sync_copy(rows2, s2_hbm.at[pl.ds(base, CHUNK)])
            return carry

        lax.fori_loop(0, n_chunks, body, 0)

    return gather_k


def _make_scatter(n_edge, n_mesh_pad, n_chunks):
    rows_per_tile = n_mesh_pad // NS

    @functools.partial(
        pl.kernel,
        out_type=jax.ShapeDtypeStruct((NC, n_mesh_pad, HIDDEN), jnp.float32),
        mesh=_sc_mesh(),
        scratch_types=[
            pltpu.VMEM((CHUNK,), jnp.int32),
            pltpu.VMEM((CHUNK, HIDDEN), jnp.float32),
            pltpu.VMEM_SHARED((n_mesh_pad, HIDDEN), jnp.float32),
        ],
    )
    def scatter_k(e_hbm, dsti_hbm, zeros_hbm, out_hbm, idx, rows, acc):
        cid = lax.axis_index("c")
        sid = lax.axis_index("s")
        wid = sid * NC + cid
        base0 = wid * (n_chunks * CHUNK)
        my_rows = pl.ds(sid * rows_per_tile, rows_per_tile)

        pltpu.sync_copy(zeros_hbm, acc.at[my_rows])
        plsc.subcore_barrier()

        def body(i, carry):
            base = base0 + i * CHUNK
            pltpu.sync_copy(dsti_hbm.at[pl.ds(base, CHUNK)], idx)
            pltpu.sync_copy(e_hbm.at[pl.ds(base, CHUNK)], rows)
            pltpu.sync_copy(rows, acc.at[idx], add=True)
            return carry

        lax.fori_loop(0, n_chunks, body, 0)
        plsc.subcore_barrier()

        pltpu.sync_copy(acc.at[my_rows], out_hbm.at[cid, my_rows])

    return scatter_k


def kernel(grid_node_features, mesh_node_features, grid2mesh_edge_features,
           grid2mesh_edge_indices_src, grid2mesh_edge_indices_dst,
           We1, be1, We2, be2, ge, bbe,
           Wn1, bn1, Wn2, bn2, gn, bbn,
           Wg1, bg1, Wg2, bg2, gg, bbg):
    n_grid, hid = grid_node_features.shape
    n_mesh = mesh_node_features.shape[0]
    n_edge = grid2mesh_edge_features.shape[0]
    assert hid == HIDDEN
    assert n_edge % (NW * CHUNK) == 0 and n_mesh % NS == 0
    n_chunks = n_edge // (NW * CHUNK)

    We1_e, We1_s, We1_d = We1[:hid], We1[hid:2 * hid], We1[2 * hid:]
    Wn1_a, Wn1_m = Wn1[:hid], Wn1[hid:]
    r = lambda v: v.reshape(1, hid)

    br_g = 2000
    psrc, grid_out = pl.pallas_call(
        _grid_body,
        grid=(n_grid // br_g,),
        in_specs=[_rows(br_g), _full((hid, hid)), _full((hid, hid)),
                  _full((1, hid)), _full((hid, hid)), _full((1, hid)),
                  _full((1, hid)), _full((1, hid))],
        out_specs=[_rows(br_g), _rows(br_g)],
        out_shape=[jax.ShapeDtypeStruct((n_grid, hid), jnp.float32),
                   jax.ShapeDtypeStruct((n_grid, hid), jnp.float32)],
    )(grid_node_features, We1_s, Wg1, r(bg1), Wg2, r(bg2), r(gg), r(bbg))

    br_m = 2000
    pdst = pl.pallas_call(
        _proj_body,
        grid=(n_mesh // br_m,),
        in_specs=[_rows(br_m), _full((hid, hid))],
        out_specs=_rows(br_m),
        out_shape=jax.ShapeDtypeStruct((n_mesh, hid), jnp.float32),
    )(mesh_node_features, We1_d)

    s1, s2 = _make_gather(n_edge, n_chunks)(
        psrc, pdst, grid2mesh_edge_indices_src, grid2mesh_edge_indices_dst)

    br_e = 2000
    e_feats = pl.pallas_call(
        _edge_body,
        grid=(n_edge // br_e,),
        in_specs=[_rows(br_e), _rows(br_e), _rows(br_e), _full((hid, hid)),
                  _full((1, hid)), _full((hid, hid)), _full((1, hid)),
                  _full((1, hid)), _full((1, hid))],
        out_specs=_rows(br_e),
        out_shape=jax.ShapeDtypeStruct((n_edge, hid), jnp.float32),
    )(grid2mesh_edge_features, s1, s2, We1_e, r(be1), We2, r(be2), r(ge),
      r(bbe))

    n_mesh_pad = ((n_mesh + NS * 8 - 1) // (NS * 8)) * NS * 8
    zeros_tile = jnp.zeros((n_mesh_pad // NS, hid), jnp.float32)
    partials = _make_scatter(n_edge, n_mesh_pad, n_chunks)(
        e_feats, grid2mesh_edge_indices_dst, zeros_tile)
    p0 = partials[0, :n_mesh]
    p1 = partials[1, :n_mesh]

    br_n = 2000
    mesh_out = pl.pallas_call(
        _node_body,
        grid=(n_mesh // br_n,),
        in_specs=[_rows(br_n), _rows(br_n), _rows(br_n), _full((hid, hid)),
                  _full((hid, hid)), _full((1, hid)), _full((hid, hid)),
                  _full((1, hid)), _full((1, hid)), _full((1, hid))],
        out_specs=_rows(br_n),
        out_shape=jax.ShapeDtypeStruct((n_mesh, hid), jnp.float32),
    )(p0, p1, mesh_node_features, Wn1_a, Wn1_m, r(bn1), Wn2,
      r(bn2), r(gn), r(bbn))

    return (grid_out, mesh_out)


# speedup vs baseline: 4.0954x; 1.5108x over previous
import functools

import jax
import jax.numpy as jnp
from jax import lax
from jax.experimental import pallas as pl
from jax.experimental.pallas import tpu as pltpu
from jax.experimental.pallas import tpu_sc as plsc

HIDDEN = 128
NC = 2
NS = 16
NW = NC * NS
CHUNK = 80


def _ln(h, g, b):
    mu = jnp.mean(h, axis=-1, keepdims=True)
    d = h - mu
    var = jnp.mean(d * d, axis=-1, keepdims=True)
    return d * lax.rsqrt(var + 1e-5) * g + b


def _dot(a, b):
    return jnp.dot(a, b, preferred_element_type=jnp.float32)


def _gridmlp_body(x_ref, wg1_ref, bg1_ref, wg2_ref, bg2_ref, gg_ref,
                  bbg_ref, out_ref):
    x = x_ref[...]
    h = _dot(x, wg1_ref[...]) + bg1_ref[...]
    h = h * jax.nn.sigmoid(h)
    h2 = _dot(h, wg2_ref[...]) + bg2_ref[...]
    out_ref[...] = x + _ln(h2, gg_ref[...], bbg_ref[...])


def _proj_body(m_ref, we1d_ref, p_ref):
    p_ref[...] = _dot(m_ref[...], we1d_ref[...])


def _edge_body(e_ref, s1_ref, s2_ref, we1e_ref, be1_ref, we2_ref, be2_ref,
               ge_ref, bbe_ref, out_ref):
    e = e_ref[...]
    h = _dot(e, we1e_ref[...]) + s1_ref[...] + s2_ref[...] + be1_ref[...]
    h = h * jax.nn.sigmoid(h)
    h2 = _dot(h, we2_ref[...]) + be2_ref[...]
    out_ref[...] = e + _ln(h2, ge_ref[...], bbe_ref[...])


def _node_body(a1_ref, a2_ref, m_ref, wn1a_ref, wn1m_ref, bn1_ref, wn2_ref,
               bn2_ref, gn_ref, bbn_ref, out_ref):
    m = m_ref[...]
    agg = a1_ref[...] + a2_ref[...]
    h = _dot(agg, wn1a_ref[...]) + _dot(m, wn1m_ref[...]) + bn1_ref[...]
    h = h * jax.nn.sigmoid(h)
    h2 = _dot(h, wn2_ref[...]) + bn2_ref[...]
    out_ref[...] = m + _ln(h2, gn_ref[...], bbn_ref[...])


def _full(shape):
    return pl.BlockSpec(shape, lambda i: (0,) * len(shape))


def _rows(br):
    return pl.BlockSpec((br, HIDDEN), lambda i: (i, 0))


def _sc_mesh():
    return plsc.VectorSubcoreMesh(core_axis_name="c", subcore_axis_name="s",
                                  num_cores=NC, num_subcores=NS)


NB = 5


def _make_gather(n_edge, n_chunks):
    epw = n_chunks * CHUNK
    n_groups = n_chunks // NB
    assert n_chunks % NB == 0

    @functools.partial(
        pl.kernel,
        out_type=[jax.ShapeDtypeStruct((n_edge, HIDDEN), jnp.float32),
                  jax.ShapeDtypeStruct((n_edge, HIDDEN), jnp.float32)],
        mesh=_sc_mesh(),
        scratch_types=[
            pltpu.VMEM((epw,), jnp.int32),
            pltpu.VMEM((epw,), jnp.int32),
            pltpu.VMEM((NB * CHUNK, HIDDEN), jnp.float32),
            pltpu.VMEM((NB * CHUNK, HIDDEN), jnp.float32),
        ] + [pltpu.SemaphoreType.DMA] * (2 * NB),
    )
    def gather_k(psrc_hbm, pdst_hbm, srci_hbm, dsti_hbm, s1_hbm, s2_hbm,
                 idx1, idx2, rows1, rows2, *sems):
        wid = lax.axis_index("s") * NC + lax.axis_index("c")
        base0 = wid * epw

        pltpu.sync_copy(srci_hbm.at[pl.ds(base0, epw)], idx1)
        pltpu.sync_copy(dsti_hbm.at[pl.ds(base0, epw)], idx2)

        def fire(c, b):
            csl = pl.ds(c * CHUNK, CHUNK)
            bsl = pl.ds(b * CHUNK, CHUNK)
            pltpu.async_copy(psrc_hbm.at[idx1.at[csl]], rows1.at[bsl], sems[b])
            pltpu.async_copy(pdst_hbm.at[idx2.at[csl]], rows2.at[bsl],
                             sems[NB + b])

        for b in range(NB):
            fire(b, b)

        def body(g, carry):
            for b in range(NB):
                c = g * NB + b
                bsl = pl.ds(b * CHUNK, CHUNK)
                hb = pl.ds(base0 + c * CHUNK, CHUNK)
                csl = pl.ds(c * CHUNK, CHUNK)
                pltpu.make_async_copy(psrc_hbm.at[idx1.at[csl]],
                                      rows1.at[bsl], sems[b]).wait()
                pltpu.make_async_copy(pdst_hbm.at[idx2.at[csl]],
                                      rows2.at[bsl], sems[NB + b]).wait()
                pltpu.async_copy(rows1.at[bsl], s1_hbm.at[hb], sems[b])
                pltpu.async_copy(rows2.at[bsl], s2_hbm.at[hb], sems[NB + b])
                pltpu.make_async_copy(rows1.at[bsl], s1_hbm.at[hb],
                                      sems[b]).wait()
                pltpu.make_async_copy(rows2.at[bsl], s2_hbm.at[hb],
                                      sems[NB + b]).wait()

                @pl.when(g < n_groups - 1)
                def _():
                    fire(c + NB, b)
            return carry

        lax.fori_loop(0, n_groups, body, 0)

    return gather_k


def _make_scatter(n_edge, n_mesh_pad, n_chunks):
    rows_per_tile = n_mesh_pad // NS
    nbs = 3
    n_groups = (n_chunks + nbs - 1) // nbs

    @functools.partial(
        pl.kernel,
        out_type=jax.ShapeDtypeStruct((NC, n_mesh_pad, HIDDEN), jnp.float32),
        mesh=_sc_mesh(),
        scratch_types=[
            pltpu.VMEM((n_chunks, CHUNK), jnp.int32),
            pltpu.VMEM((nbs * CHUNK, HIDDEN), jnp.float32),
            pltpu.VMEM_SHARED((n_mesh_pad, HIDDEN), jnp.float32),
        ] + [pltpu.SemaphoreType.DMA] * nbs,
    )
    def scatter_k(e_hbm, dsti3_hbm, zeros_hbm, out_hbm, idxall, rows, acc,
                  *sems):
        cid = lax.axis_index("c")
        sid = lax.axis_index("s")
        wid = sid * NC + cid
        base0 = wid * (n_chunks * CHUNK)
        my_rows = pl.ds(sid * rows_per_tile, rows_per_tile)

        pltpu.sync_copy(dsti3_hbm.at[wid], idxall)

        def fire(c, b):
            pltpu.async_copy(e_hbm.at[pl.ds(base0 + c * CHUNK, CHUNK)],
                             rows.at[pl.ds(b * CHUNK, CHUNK)], sems[b])

        for b in range(nbs):
            fire(b, b)

        pltpu.sync_copy(zeros_hbm, acc.at[my_rows])
        plsc.subcore_barrier()

        def body(g, carry):
            for b in range(nbs):
                c = g * nbs + b
                bsl = pl.ds(b * CHUNK, CHUNK)

                @pl.when(c < n_chunks)
                def _():
                    pltpu.make_async_copy(
                        e_hbm.at[pl.ds(base0 + c * CHUNK, CHUNK)],
                        rows.at[bsl], sems[b]).wait()
                    pltpu.sync_copy(rows.at[bsl], acc.at[idxall.at[c]],
                                    add=True)

                @pl.when(c + nbs < n_chunks)
                def _():
                    fire(c + nbs, b)
            return carry

        lax.fori_loop(0, n_groups, body, 0)
        plsc.subcore_barrier()

        pltpu.sync_copy(acc.at[my_rows], out_hbm.at[cid, my_rows])

    return scatter_k


def kernel(grid_node_features, mesh_node_features, grid2mesh_edge_features,
           grid2mesh_edge_indices_src, grid2mesh_edge_indices_dst,
           We1, be1, We2, be2, ge, bbe,
           Wn1, bn1, Wn2, bn2, gn, bbn,
           Wg1, bg1, Wg2, bg2, gg, bbg):
    n_grid, hid = grid_node_features.shape
    n_mesh = mesh_node_features.shape[0]
    n_edge = grid2mesh_edge_features.shape[0]
    assert hid == HIDDEN
    assert n_edge % (NW * CHUNK) == 0 and n_mesh % NS == 0
    n_chunks = n_edge // (NW * CHUNK)

    We1_e, We1_s, We1_d = We1[:hid], We1[hid:2 * hid], We1[2 * hid:]
    Wn1_a, Wn1_m = Wn1[:hid], Wn1[hid:]
    r = lambda v: v.reshape(1, hid)

    br_g = 2000
    psrc = pl.pallas_call(
        _proj_body,
        grid=(n_grid // br_g,),
        in_specs=[_rows(br_g), _full((hid, hid))],
        out_specs=_rows(br_g),
        out_shape=jax.ShapeDtypeStruct((n_grid, hid), jnp.float32),
    )(grid_node_features, We1_s)

    grid_out = pl.pallas_call(
        _gridmlp_body,
        grid=(n_grid // br_g,),
        in_specs=[_rows(br_g), _full((hid, hid)), _full((1, hid)),
                  _full((hid, hid)), _full((1, hid)), _full((1, hid)),
                  _full((1, hid))],
        out_specs=_rows(br_g),
        out_shape=jax.ShapeDtypeStruct((n_grid, hid), jnp.float32),
    )(grid_node_features, Wg1, r(bg1), Wg2, r(bg2), r(gg), r(bbg))

    br_m = 2000
    pdst = pl.pallas_call(
        _proj_body,
        grid=(n_mesh // br_m,),
        in_specs=[_rows(br_m), _full((hid, hid))],
        out_specs=_rows(br_m),
        out_shape=jax.ShapeDtypeStruct((n_mesh, hid), jnp.float32),
    )(mesh_node_features, We1_d)

    s1, s2 = _make_gather(n_edge, n_chunks)(
        psrc, pdst, grid2mesh_edge_indices_src, grid2mesh_edge_indices_dst)

    br_e = 2000
    e_feats = pl.pallas_call(
        _edge_body,
        grid=(n_edge // br_e,),
        in_specs=[_rows(br_e), _rows(br_e), _rows(br_e), _full((hid, hid)),
                  _full((1, hid)), _full((hid, hid)), _full((1, hid)),
                  _full((1, hid)), _full((1, hid))],
        out_specs=_rows(br_e),
        out_shape=jax.ShapeDtypeStruct((n_edge, hid), jnp.float32),
    )(grid2mesh_edge_features, s1, s2, We1_e, r(be1), We2, r(be2), r(ge),
      r(bbe))

    n_mesh_pad = ((n_mesh + NS * 8 - 1) // (NS * 8)) * NS * 8
    zeros_tile = jnp.zeros((n_mesh_pad // NS, hid), jnp.float32)
    dsti3 = grid2mesh_edge_indices_dst.reshape(NW, n_chunks, CHUNK)
    partials = _make_scatter(n_edge, n_mesh_pad, n_chunks)(
        e_feats, dsti3, zeros_tile)
    p0 = partials[0, :n_mesh]
    p1 = partials[1, :n_mesh]

    br_n = 2000
    mesh_out = pl.pallas_call(
        _node_body,
        grid=(n_mesh // br_n,),
        in_specs=[_rows(br_n), _rows(br_n), _rows(br_n), _full((hid, hid)),
                  _full((hid, hid)), _full((1, hid)), _full((hid, hid)),
                  _full((1, hid)), _full((1, hid)), _full((1, hid))],
        out_specs=_rows(br_n),
        out_shape=jax.ShapeDtypeStruct((n_mesh, hid), jnp.float32),
    )(p0, p1, mesh_node_features, Wn1_a, Wn1_m, r(bn1), Wn2,
      r(bn2), r(gn), r(bbn))

    return (grid_out, mesh_out)
